# baseline (speedup 1.0000x reference)
import functools

import jax
import jax.numpy as jnp
from jax import lax
from jax.experimental import pallas as pl
from jax.experimental.pallas import tpu as pltpu

_BN = 2000


def _mm(a, b):
    return lax.dot_general(a, b, (((1,), (0,)), ((), ())),
                           preferred_element_type=jnp.float32,
                           precision=lax.Precision.HIGHEST)


def _body(feat_ref, aux_ref, W1_ref, vecs_ref, W2p_ref, Wsegp_ref, bias2_ref,
          out_ref, G_acc, s_acc, w1e, be, loss_acc, *, nb, n):
    i = pl.program_id(0)

    @pl.when(i == 0)
    def _init():
        G_acc[...] = jnp.zeros_like(G_acc)
        s_acc[...] = jnp.zeros_like(s_acc)
        loss_acc[...] = jnp.zeros_like(loss_acc)

    f = feat_ref[...]

    @pl.when(i < nb)
    def _phase1():
        G_acc[...] += lax.dot_general(f, f, (((0,), (0,)), ((), ())),
                                      preferred_element_type=jnp.float32,
                                      precision=lax.Precision.HIGHEST)
        s_acc[0:1, :] += jnp.sum(f, axis=0, keepdims=True)

    @pl.when(i == nb - 1)
    def _stats():
        G = G_acc[...]
        s = s_acc[0:1, :]
        W1 = W1_ref[...]
        b1 = vecs_ref[0:1, :]
        gamma = vecs_ref[1:2, :]
        beta = vecs_ref[2:3, :]
        sW = _mm(s, W1)
        mean = sW / n + b1
        GW = _mm(G, W1)
        quad = jnp.sum(W1 * GW, axis=0, keepdims=True)
        ex2 = (quad + 2.0 * b1 * sW) / n + b1 * b1
        var = ex2 - mean * mean
        scale = gamma / jnp.sqrt(var + 1e-3)
        w1e[...] = W1 * scale
        be[0:1, :] = beta + (b1 - mean) * scale

    @pl.when(i >= nb)
    def _phase2():
        bn = f.shape[0]
        lane = lax.broadcasted_iota(jnp.int32, (bn, 128), 1)
        aux = aux_ref[...]
        lg = _mm(f, Wsegp_ref[...]) + bias2_ref[1:2, :]
        m = jnp.max(lg, axis=1, keepdims=True)
        lse = jnp.log(jnp.sum(jnp.exp(lg - m), axis=1, keepdims=True)) + m
        seg = aux[:, 6:7]
        seg_i = seg.astype(jnp.int32)
        ltgt = jnp.sum(jnp.where(lane == seg_i, lg, 0.0), axis=1, keepdims=True)
        valid = (seg != -1.0).astype(jnp.float32)
        nll = (lse - ltgt) * valid
        r = jnp.maximum(_mm(f, w1e[...]) + be[0:1, :], 0.0)
        bp = _mm(r, W2p_ref[...]) + bias2_ref[0:1, :]
        px, py, pz = bp[:, 0:1], bp[:, 1:2], bp[:, 2:3]
        gx = aux[:, 3:4] - aux[:, 0:1]
        gy = aux[:, 4:5] - aux[:, 1:2]
        gz = aux[:, 5:6] - aux[:, 2:3]
        mask = (aux[:, 7:8] != -1.0).astype(jnp.float32)
        l1 = (jnp.abs(px - gx) + jnp.abs(py - gy) + jnp.abs(pz - gz)) * mask
        pn = jnp.sqrt(px * px + py * py + pz * pz) + 1e-8
        gn = jnp.sqrt(gx * gx + gy * gy + gz * gz) + 1e-8
        cos = -(px * gx + py * gy + pz * gz) / (pn * gn) * mask
        stats = (jnp.where(lane == 0, nll, 0.0)
                 + jnp.where(lane == 1, valid, 0.0)
                 + jnp.where(lane == 2, l1, 0.0)
                 + jnp.where(lane == 3, mask, 0.0)
                 + jnp.where(lane == 4, cos, 0.0))
        loss_acc[0:1, :] += jnp.sum(stats, axis=0, keepdims=True)

    @pl.when(i == 2 * nb - 1)
    def _final():
        a = loss_acc[0:1, :]
        lr = lax.broadcasted_iota(jnp.int32, (1, 128), 1)

        def pick(j):
            return jnp.sum(jnp.where(lr == j, a, 0.0))

        seg_loss = pick(0) / (pick(1) + 1e-8)
        denom = pick(3) + 1e-8
        l1_loss = pick(2) / denom
        cos_loss = pick(4) / denom
        total = seg_loss + l1_loss + cos_loss
        row = (jnp.where(lr == 0, total, 0.0)
               + jnp.where(lr == 1, seg_loss, 0.0)
               + jnp.where(lr == 2, l1_loss, 0.0)
               + jnp.where(lr == 3, cos_loss, 0.0))
        out_ref[...] = jnp.broadcast_to(row, out_ref.shape)


def kernel(feat, coord, instance_centroid, W1, b1, gamma, beta, W2, b2,
           Wseg, bseg, segment, instance):
    n, c = feat.shape
    k = Wseg.shape[1]
    bn = _BN
    assert n % bn == 0
    nb = n // bn
    aux = jnp.concatenate(
        [coord, instance_centroid,
         segment.astype(jnp.float32)[:, None],
         instance.astype(jnp.float32)[:, None]], axis=1)
    vecs = (jnp.zeros((8, c), jnp.float32)
            .at[0].set(b1).at[1].set(gamma).at[2].set(beta))
    W2p = jnp.zeros((c, 128), jnp.float32).at[:, :3].set(W2)
    Wsegp = jnp.zeros((c, 128), jnp.float32).at[:, :k].set(Wseg)
    bias2 = (jnp.full((8, 128), 0.0, jnp.float32)
             .at[0, :3].set(b2)
             .at[1, :].set(-1e30).at[1, :k].set(bseg))

    out = pl.pallas_call(
        functools.partial(_body, nb=nb, n=float(n)),
        grid=(2 * nb,),
        in_specs=[
            pl.BlockSpec((bn, c), lambda i: (jnp.where(i < nb, i, i - nb), 0)),
            pl.BlockSpec((bn, 8), lambda i: (jnp.where(i < nb, 0, i - nb), 0)),
            pl.BlockSpec((c, c), lambda i: (0, 0)),
            pl.BlockSpec((8, c), lambda i: (0, 0)),
            pl.BlockSpec((c, 128), lambda i: (0, 0)),
            pl.BlockSpec((c, 128), lambda i: (0, 0)),
            pl.BlockSpec((8, 128), lambda i: (0, 0)),
        ],
        out_specs=pl.BlockSpec((8, 128), lambda i: (0, 0)),
        out_shape=jax.ShapeDtypeStruct((8, 128), jnp.float32),
        scratch_shapes=[
            pltpu.VMEM((c, c), jnp.float32),
            pltpu.VMEM((8, c), jnp.float32),
            pltpu.VMEM((c, c), jnp.float32),
            pltpu.VMEM((8, c), jnp.float32),
            pltpu.VMEM((8, 128), jnp.float32),
        ],
    )(feat, aux, W1, vecs, W2p, Wsegp, bias2)
    return (out[0, 0], out[0, 1], out[0, 2], out[0, 3])


# speedup vs baseline: 1.1210x; 1.1210x over previous
import functools

import jax
import jax.numpy as jnp
from jax import lax
from jax.experimental import pallas as pl
from jax.experimental.pallas import tpu as pltpu

_BN = 2000


def _mm(a, b):
    return lax.dot_general(a, b, (((1,), (0,)), ((), ())),
                           preferred_element_type=jnp.float32,
                           precision=lax.Precision.DEFAULT)


def _body(feat_ref, aux_ref, W1_ref, vecs_ref, W2p_ref, Wsegp_ref, bias2_ref,
          out_ref, G_acc, s_acc, w1e, be, loss_acc, *, nb, n):
    i = pl.program_id(0)

    @pl.when(i == 0)
    def _init():
        G_acc[...] = jnp.zeros_like(G_acc)
        s_acc[...] = jnp.zeros_like(s_acc)
        loss_acc[...] = jnp.zeros_like(loss_acc)

    f = feat_ref[...]

    @pl.when(i < nb)
    def _phase1():
        G_acc[...] += lax.dot_general(f, f, (((0,), (0,)), ((), ())),
                                      preferred_element_type=jnp.float32,
                                      precision=lax.Precision.DEFAULT)
        s_acc[0:1, :] += jnp.sum(f, axis=0, keepdims=True)

    @pl.when(i == nb - 1)
    def _stats():
        G = G_acc[...]
        s = s_acc[0:1, :]
        W1 = W1_ref[...]
        b1 = vecs_ref[0:1, :]
        gamma = vecs_ref[1:2, :]
        beta = vecs_ref[2:3, :]
        sW = _mm(s, W1)
        mean = sW / n + b1
        GW = _mm(G, W1)
        quad = jnp.sum(W1 * GW, axis=0, keepdims=True)
        ex2 = (quad + 2.0 * b1 * sW) / n + b1 * b1
        var = ex2 - mean * mean
        scale = gamma / jnp.sqrt(var + 1e-3)
        w1e[...] = W1 * scale
        be[0:1, :] = beta + (b1 - mean) * scale

    @pl.when(i >= nb)
    def _phase2():
        bn = f.shape[0]
        lane = lax.broadcasted_iota(jnp.int32, (bn, 128), 1)
        aux = aux_ref[...]
        lg = _mm(f, Wsegp_ref[...]) + bias2_ref[1:2, :]
        m = jnp.max(lg, axis=1, keepdims=True)
        lse = jnp.log(jnp.sum(jnp.exp(lg - m), axis=1, keepdims=True)) + m
        seg = aux[:, 6:7]
        seg_i = seg.astype(jnp.int32)
        ltgt = jnp.sum(jnp.where(lane == seg_i, lg, 0.0), axis=1, keepdims=True)
        valid = (seg != -1.0).astype(jnp.float32)
        nll = (lse - ltgt) * valid
        r = jnp.maximum(_mm(f, w1e[...]) + be[0:1, :], 0.0)
        bp = _mm(r, W2p_ref[...]) + bias2_ref[0:1, :]
        px, py, pz = bp[:, 0:1], bp[:, 1:2], bp[:, 2:3]
        gx = aux[:, 3:4] - aux[:, 0:1]
        gy = aux[:, 4:5] - aux[:, 1:2]
        gz = aux[:, 5:6] - aux[:, 2:3]
        mask = (aux[:, 7:8] != -1.0).astype(jnp.float32)
        l1 = (jnp.abs(px - gx) + jnp.abs(py - gy) + jnp.abs(pz - gz)) * mask
        pn = jnp.sqrt(px * px + py * py + pz * pz) + 1e-8
        gn = jnp.sqrt(gx * gx + gy * gy + gz * gz) + 1e-8
        cos = -(px * gx + py * gy + pz * gz) / (pn * gn) * mask
        stats = (jnp.where(lane == 0, nll, 0.0)
                 + jnp.where(lane == 1, valid, 0.0)
                 + jnp.where(lane == 2, l1, 0.0)
                 + jnp.where(lane == 3, mask, 0.0)
                 + jnp.where(lane == 4, cos, 0.0))
        loss_acc[0:1, :] += jnp.sum(stats, axis=0, keepdims=True)

    @pl.when(i == 2 * nb - 1)
    def _final():
        a = loss_acc[0:1, :]
        lr = lax.broadcasted_iota(jnp.int32, (1, 128), 1)

        def pick(j):
            return jnp.sum(jnp.where(lr == j, a, 0.0))

        seg_loss = pick(0) / (pick(1) + 1e-8)
        denom = pick(3) + 1e-8
        l1_loss = pick(2) / denom
        cos_loss = pick(4) / denom
        total = seg_loss + l1_loss + cos_loss
        row = (jnp.where(lr == 0, total, 0.0)
               + jnp.where(lr == 1, seg_loss, 0.0)
               + jnp.where(lr == 2, l1_loss, 0.0)
               + jnp.where(lr == 3, cos_loss, 0.0))
        out_ref[...] = jnp.broadcast_to(row, out_ref.shape)


def kernel(feat, coord, instance_centroid, W1, b1, gamma, beta, W2, b2,
           Wseg, bseg, segment, instance):
    n, c = feat.shape
    k = Wseg.shape[1]
    bn = _BN
    assert n % bn == 0
    nb = n // bn
    aux = jnp.concatenate(
        [coord, instance_centroid,
         segment.astype(jnp.float32)[:, None],
         instance.astype(jnp.float32)[:, None]], axis=1)
    vecs = (jnp.zeros((8, c), jnp.float32)
            .at[0].set(b1).at[1].set(gamma).at[2].set(beta))
    W2p = jnp.zeros((c, 128), jnp.float32).at[:, :3].set(W2)
    Wsegp = jnp.zeros((c, 128), jnp.float32).at[:, :k].set(Wseg)
    bias2 = (jnp.full((8, 128), 0.0, jnp.float32)
             .at[0, :3].set(b2)
             .at[1, :].set(-1e30).at[1, :k].set(bseg))

    out = pl.pallas_call(
        functools.partial(_body, nb=nb, n=float(n)),
        grid=(2 * nb,),
        in_specs=[
            pl.BlockSpec((bn, c), lambda i: (jnp.where(i < nb, i, i - nb), 0)),
            pl.BlockSpec((bn, 8), lambda i: (jnp.where(i < nb, 0, i - nb), 0)),
            pl.BlockSpec((c, c), lambda i: (0, 0)),
            pl.BlockSpec((8, c), lambda i: (0, 0)),
            pl.BlockSpec((c, 128), lambda i: (0, 0)),
            pl.BlockSpec((c, 128), lambda i: (0, 0)),
            pl.BlockSpec((8, 128), lambda i: (0, 0)),
        ],
        out_specs=pl.BlockSpec((8, 128), lambda i: (0, 0)),
        out_shape=jax.ShapeDtypeStruct((8, 128), jnp.float32),
        scratch_shapes=[
            pltpu.VMEM((c, c), jnp.float32),
            pltpu.VMEM((8, c), jnp.float32),
            pltpu.VMEM((c, c), jnp.float32),
            pltpu.VMEM((8, c), jnp.float32),
            pltpu.VMEM((8, 128), jnp.float32),
        ],
    )(feat, aux, W1, vecs, W2p, Wsegp, bias2)
    return (out[0, 0], out[0, 1], out[0, 2], out[0, 3])


# speedup vs baseline: 4.4652x; 3.9831x over previous
import functools

import jax
import jax.numpy as jnp
from jax import lax
from jax.experimental import pallas as pl
from jax.experimental.pallas import tpu as pltpu

_BN = 8000


def _dot(a, b, dims):
    return lax.dot_general(a, b, (dims, ((), ())),
                           preferred_element_type=jnp.float32,
                           precision=lax.Precision.DEFAULT)


def _body(feat_ref, auxT_ref, W1_ref, vecs_ref, W2T8_ref, WsegT_ref,
          bcols_ref, out_ref, G_acc, s_acc, w1e, be_col, loss_acc, *, nb, n):
    i = pl.program_id(0)
    bn = feat_ref.shape[0]

    @pl.when(i == 0)
    def _init():
        G_acc[...] = jnp.zeros_like(G_acc)
        s_acc[...] = jnp.zeros_like(s_acc)
        loss_acc[...] = jnp.zeros_like(loss_acc)

    f = feat_ref[...]

    @pl.when(i < nb)
    def _phase1():
        G_acc[...] += _dot(f, f, ((0,), (0,)))
        s_acc[0:1, :] += jnp.sum(f, axis=0, keepdims=True)

    @pl.when(i == nb - 1)
    def _stats():
        G = G_acc[...]
        s = s_acc[0:1, :]
        W1 = W1_ref[...]
        b1 = vecs_ref[0:1, :]
        gamma = vecs_ref[1:2, :]
        beta = vecs_ref[2:3, :]
        sW = _dot(s, W1, ((1,), (0,)))
        mean = sW / n + b1
        GW = _dot(G, W1, ((1,), (0,)))
        quad = jnp.sum(W1 * GW, axis=0, keepdims=True)
        ex2 = (quad + 2.0 * b1 * sW) / n + b1 * b1
        var = ex2 - mean * mean
        scale = gamma / jnp.sqrt(var + 1e-3)
        w1e[...] = W1 * scale
        be_row = beta + (b1 - mean) * scale
        c = W1.shape[0]
        eye = (lax.broadcasted_iota(jnp.int32, (c, c), 0)
               == lax.broadcasted_iota(jnp.int32, (c, c), 1)).astype(jnp.float32)
        be_col[:, 0:1] = _dot(eye, be_row, ((1,), (1,)))

    @pl.when(i >= nb)
    def _phase2():
        auxT = auxT_ref[...].reshape(8, bn)
        lgT = _dot(WsegT_ref[...], f, ((1,), (1,))) + bcols_ref[:, 0:1]
        m = jnp.max(lgT, axis=0, keepdims=True)
        S = jnp.sum(jnp.exp(lgT - m), axis=0, keepdims=True)
        lse = jnp.log(S) + m
        segT = auxT[6:7, :]
        cls = lax.broadcasted_iota(jnp.int32, lgT.shape, 0)
        ltgt = jnp.sum(jnp.where(cls == segT.astype(jnp.int32), lgT, 0.0),
                       axis=0, keepdims=True)
        valid = (segT != -1.0).astype(jnp.float32)
        nll = (lse - ltgt) * valid
        hT = _dot(w1e[...], f, ((0,), (1,)))
        rT = jnp.maximum(hT + be_col[:, 0:1], 0.0)
        bpT = _dot(W2T8_ref[...], rT, ((1,), (0,))) + bcols_ref[0:8, 1:2]
        px, py, pz = bpT[0:1, :], bpT[1:2, :], bpT[2:3, :]
        gx = auxT[3:4, :] - auxT[0:1, :]
        gy = auxT[4:5, :] - auxT[1:2, :]
        gz = auxT[5:6, :] - auxT[2:3, :]
        mask = (auxT[7:8, :] != -1.0).astype(jnp.float32)
        l1 = (jnp.abs(px - gx) + jnp.abs(py - gy) + jnp.abs(pz - gz)) * mask
        pn = jnp.sqrt(px * px + py * py + pz * pz) + 1e-8
        gn = jnp.sqrt(gx * gx + gy * gy + gz * gz) + 1e-8
        cos = -(px * gx + py * gy + pz * gz) / (pn * gn) * mask
        riota = lax.broadcasted_iota(jnp.int32, (8, bn), 0)
        rows = (jnp.where(riota == 0, nll, 0.0)
                + jnp.where(riota == 1, valid, 0.0)
                + jnp.where(riota == 2, l1, 0.0)
                + jnp.where(riota == 3, mask, 0.0)
                + jnp.where(riota == 4, cos, 0.0))
        loss_acc[...] += rows

    @pl.when(i == 2 * nb - 1)
    def _final():
        ones = jnp.ones((1, bn), jnp.float32)
        sums = _dot(loss_acc[...], ones, ((1,), (1,)))
        r8 = lax.broadcasted_iota(jnp.int32, (8, 1), 0)

        def pick(j):
            return jnp.sum(jnp.where(r8 == j, sums, 0.0))

        seg_loss = pick(0) / (pick(1) + 1e-8)
        denom = pick(3) + 1e-8
        l1_loss = pick(2) / denom
        cos_loss = pick(4) / denom
        total = seg_loss + l1_loss + cos_loss
        lr = lax.broadcasted_iota(jnp.int32, (1, 128), 1)
        row = (jnp.where(lr == 0, total, 0.0)
               + jnp.where(lr == 1, seg_loss, 0.0)
               + jnp.where(lr == 2, l1_loss, 0.0)
               + jnp.where(lr == 3, cos_loss, 0.0))
        out_ref[...] = jnp.broadcast_to(row, out_ref.shape)


def kernel(feat, coord, instance_centroid, W1, b1, gamma, beta, W2, b2,
           Wseg, bseg, segment, instance):
    n, c = feat.shape
    k = Wseg.shape[1]
    bn = _BN
    assert n % bn == 0
    nb = n // bn
    auxT = jnp.concatenate(
        [coord.T, instance_centroid.T,
         segment.astype(jnp.float32)[None, :],
         instance.astype(jnp.float32)[None, :]], axis=0)
    aux3 = auxT.reshape(8, nb, bn).transpose(1, 0, 2)
    vecs = (jnp.zeros((8, c), jnp.float32)
            .at[0].set(b1).at[1].set(gamma).at[2].set(beta))
    W2T8 = jnp.zeros((8, c), jnp.float32).at[:3].set(W2.T)
    WsegT = jnp.zeros((32, c), jnp.float32).at[:k].set(Wseg.T)
    bcols = (jnp.zeros((32, 128), jnp.float32)
             .at[:, 0].set(-1e30).at[:k, 0].set(bseg)
             .at[:3, 1].set(b2))

    out = pl.pallas_call(
        functools.partial(_body, nb=nb, n=float(n)),
        grid=(2 * nb,),
        in_specs=[
            pl.BlockSpec((bn, c), lambda i: (jnp.where(i < nb, i, i - nb), 0)),
            pl.BlockSpec((1, 8, bn), lambda i: (jnp.where(i < nb, 0, i - nb), 0, 0)),
            pl.BlockSpec((c, c), lambda i: (0, 0)),
            pl.BlockSpec((8, c), lambda i: (0, 0)),
            pl.BlockSpec((8, c), lambda i: (0, 0)),
            pl.BlockSpec((32, c), lambda i: (0, 0)),
            pl.BlockSpec((32, 128), lambda i: (0, 0)),
        ],
        out_specs=pl.BlockSpec((8, 128), lambda i: (0, 0)),
        out_shape=jax.ShapeDtypeStruct((8, 128), jnp.float32),
        scratch_shapes=[
            pltpu.VMEM((c, c), jnp.float32),
            pltpu.VMEM((8, c), jnp.float32),
            pltpu.VMEM((c, c), jnp.float32),
            pltpu.VMEM((c, 128), jnp.float32),
            pltpu.VMEM((8, bn), jnp.float32),
        ],
    )(feat, aux3, W1, vecs, W2T8, WsegT, bcols)
    return (out[0, 0], out[0, 1], out[0, 2], out[0, 3])
